# baseline (speedup 1.0000x reference)
import functools

import jax
import jax.numpy as jnp
from jax import lax
from jax.experimental import pallas as pl
from jax.experimental.pallas import tpu as pltpu
from jax.experimental.pallas import tpu_sc as plsc

_N = 10000
_E = 160000
_F = 256
_NP = 10240
_EP = 163840
_CH = 128
_EC = 64
_NCH = _EP // 16 // _EC
_WCH = 32
_BR = 256

_mesh = plsc.VectorSubcoreMesh(core_axis_name="c", subcore_axis_name="s")


@functools.partial(
    pl.kernel,
    out_type=jax.ShapeDtypeStruct((2, _NP), jnp.float32),
    mesh=_mesh,
    scratch_types=[
        pltpu.VMEM((80, _EC), jnp.int32),
        pltpu.VMEM((_CH,), jnp.float32),
        pltpu.VMEM((_CH,), jnp.float32),
        pltpu.VMEM_SHARED((_NP,), jnp.float32),
    ],
)
def _deg_kernel(dst_hbm, out_hbm, idx_v, ones_v, zero_v, acc):
    cid = lax.axis_index("c")
    sid = lax.axis_index("s")
    for k in range(8):
        ones_v[pl.ds(k * 16, 16)] = jnp.ones((16,), jnp.float32)
        zero_v[pl.ds(k * 16, 16)] = jnp.zeros((16,), jnp.float32)
    base = sid * (_NP // 16)
    for j in range(_NP // 16 // _CH):
        pltpu.sync_copy(zero_v, acc.at[pl.ds(base + j * _CH, _CH)])
    plsc.subcore_barrier()
    pltpu.sync_copy(dst_hbm.at[sid, pl.ds(cid * 80, 80)], idx_v)

    @pl.loop(0, 80)
    def _(g):
        pltpu.sync_copy(ones_v.at[pl.ds(0, _EC)], acc.at[idx_v.at[g]],
                        add=True)

    plsc.subcore_barrier()
    pltpu.sync_copy(acc.at[pl.ds(base, _NP // 16)],
                    out_hbm.at[cid, pl.ds(base, _NP // 16)])


@functools.partial(
    pl.kernel,
    out_type=jax.ShapeDtypeStruct((2, _NP, 128), jnp.float32),
    mesh=_mesh,
    scratch_types=[
        pltpu.VMEM((_WCH, _EC), jnp.int32),
        pltpu.VMEM((_WCH, _EC), jnp.int32),
        pltpu.VMEM((_EC, 128), jnp.float32),
        pltpu.VMEM((_EC, 128), jnp.float32),
        pltpu.VMEM((_EC, 128), jnp.float32),
        pltpu.VMEM((_EC, 128), jnp.float32),
        pltpu.VMEM_SHARED((_NP, 128), jnp.float32),
        pltpu.SemaphoreType.DMA,
        pltpu.SemaphoreType.DMA,
        pltpu.SemaphoreType.DMA,
        pltpu.SemaphoreType.DMA,
    ],
)
def _conv_kernel(tbl_hbm, src_hbm, dst_hbm, out_hbm, src_w, dst_w, rb0, rb1,
                 rb2, rb3, acc, sm0, sm1, sm2, sm3):
    cid = lax.axis_index("c")
    sid = lax.axis_index("s")
    rows_v = (rb0, rb1, rb2, rb3)
    sems = (sm0, sm1, sm2, sm3)
    rows_per = _NP // 16
    base = sid * rows_per

    @pl.loop(0, _EC)
    def _(i):
        for k in range(8):
            rows_v[0][i, pl.ds(k * 16, 16)] = jnp.zeros((16,), jnp.float32)

    for j in range(rows_per // _EC):
        pltpu.sync_copy(rows_v[0], acc.at[pl.ds(base + j * _EC, _EC)])
    plsc.subcore_barrier()

    tbl = tbl_hbm.at[cid]

    @pl.loop(0, _NCH // _WCH)
    def _(w):
        pltpu.sync_copy(src_hbm.at[sid, pl.ds(w * _WCH, _WCH)], src_w)
        pltpu.sync_copy(dst_hbm.at[sid, pl.ds(w * _WCH, _WCH)], dst_w)

        for i in range(4):
            pltpu.async_copy(tbl.at[src_w.at[i]], rows_v[i], sems[i])

        @pl.loop(0, _WCH, step=4)
        def _(g):
            for i in range(4):
                pltpu.make_async_copy(tbl.at[src_w.at[g + i]], rows_v[i],
                                      sems[i]).wait()
                pltpu.sync_copy(rows_v[i], acc.at[dst_w.at[g + i]], add=True)
                gnext = jnp.minimum(g + 4 + i, _WCH - 1)
                pltpu.async_copy(tbl.at[src_w.at[gnext]], rows_v[i], sems[i])

        for i in range(4):
            pltpu.make_async_copy(tbl.at[src_w.at[_WCH - 1]], rows_v[i],
                                  sems[i]).wait()

    plsc.subcore_barrier()
    pltpu.sync_copy(acc.at[pl.ds(base, rows_per)],
                    out_hbm.at[cid, pl.ds(base, rows_per)])


def _colvec(row, n):
    eye = (lax.broadcasted_iota(jnp.int32, (n, n), 0)
           == lax.broadcasted_iota(jnp.int32, (n, n), 1)).astype(jnp.float32)
    return lax.dot_general(eye, row, (((1,), (1,)), ((), ())),
                           preferred_element_type=jnp.float32)


def _pre_body(x_ref, w_ref, deg_ref, zt_ref, dinv_ref):
    i = pl.program_id(0)
    y = jnp.dot(x_ref[...], w_ref[...], preferred_element_type=jnp.float32)
    dsl = deg_ref[:, pl.ds(i * _BR, _BR)]
    dl = lax.rsqrt(dsl[0:1] + dsl[1:2] + 1.0)
    dc = _colvec(dl, _BR)
    z = y * dc
    zt_ref[0] = z[:, :128]
    zt_ref[1] = z[:, 128:]
    dinv_ref[...] = jnp.broadcast_to(dc, (_BR, 128))


def _pre(xp, W, deg2):
    return pl.pallas_call(
        _pre_body,
        grid=(_NP // _BR,),
        in_specs=[
            pl.BlockSpec((_BR, _F), lambda i: (i, 0)),
            pl.BlockSpec((_F, _F), lambda i: (0, 0)),
            pl.BlockSpec((2, _NP), lambda i: (0, 0)),
        ],
        out_specs=[
            pl.BlockSpec((2, _BR, 128), lambda i: (0, i, 0)),
            pl.BlockSpec((_BR, 128), lambda i: (i, 0)),
        ],
        out_shape=[
            jax.ShapeDtypeStruct((2, _NP, 128), jnp.float32),
            jax.ShapeDtypeStruct((_NP, 128), jnp.float32),
        ],
    )(xp, W, deg2)


def _mid_body(res, acc_ref, ztp_ref, dinv_ref, b_ref, w_ref, *rest):
    if res:
        res_ref, zt_ref, y_ref = rest
    else:
        zt_ref, y_ref = rest
    dinv = dinv_ref[...]
    h0 = (acc_ref[0] + ztp_ref[0]) * dinv
    h1 = (acc_ref[1] + ztp_ref[1]) * dinv
    y = jnp.concatenate([h0, h1], axis=1) + b_ref[...]
    if res:
        y = y + res_ref[...]
    y = jnp.maximum(y, 0.0)
    y_ref[...] = y
    z = jnp.dot(y, w_ref[...], preferred_element_type=jnp.float32)
    zt_ref[0] = z[:, :128] * dinv
    zt_ref[1] = z[:, 128:] * dinv


def _mid(acc, ztp, dinvb, b2, W, res=None):
    in_specs = [
        pl.BlockSpec((2, _BR, 128), lambda i: (0, i, 0)),
        pl.BlockSpec((2, _BR, 128), lambda i: (0, i, 0)),
        pl.BlockSpec((_BR, 128), lambda i: (i, 0)),
        pl.BlockSpec((1, _F), lambda i: (0, 0)),
        pl.BlockSpec((_F, _F), lambda i: (0, 0)),
    ]
    args = [acc, ztp, dinvb, b2, W]
    if res is not None:
        in_specs.append(pl.BlockSpec((_BR, _F), lambda i: (i, 0)))
        args.append(res)
    return pl.pallas_call(
        functools.partial(_mid_body, res is not None),
        grid=(_NP // _BR,),
        in_specs=in_specs,
        out_specs=[
            pl.BlockSpec((2, _BR, 128), lambda i: (0, i, 0)),
            pl.BlockSpec((_BR, _F), lambda i: (i, 0)),
        ],
        out_shape=[
            jax.ShapeDtypeStruct((2, _NP, 128), jnp.float32),
            jax.ShapeDtypeStruct((_NP, _F), jnp.float32),
        ],
    )(*args)


def _final_body(acc_ref, ztp_ref, dinv_ref, b_ref, res_ref, w1_ref, b1_ref,
                w2_ref, b2_ref, w3_ref, b3_ref, out_ref):
    dinv = dinv_ref[...]
    h0 = (acc_ref[0] + ztp_ref[0]) * dinv
    h1 = (acc_ref[1] + ztp_ref[1]) * dinv
    y = jnp.concatenate([h0, h1], axis=1) + b_ref[...] + res_ref[...]
    y = jnp.maximum(y, 0.0)
    y = jnp.maximum(jnp.dot(y, w1_ref[...],
                            preferred_element_type=jnp.float32) + b1_ref[...],
                    0.0)
    y = jnp.maximum(jnp.dot(y, w2_ref[...],
                            preferred_element_type=jnp.float32) + b2_ref[...],
                    0.0)
    p = jnp.dot(y, w3_ref[...], preferred_element_type=jnp.float32) + b3_ref[...]
    col = lax.broadcasted_iota(jnp.int32, (_BR, 128), 1)
    pm = jnp.where(col < 7, p, -1e30)
    m = jnp.max(pm, axis=1, keepdims=True)
    s = jnp.sum(jnp.exp(pm - m), axis=1, keepdims=True)
    out_ref[...] = p - m - jnp.log(s)


def _final(acc, ztp, dinvb, b2d, y1, W1, b1, W2, b2, W3p, b3p):
    return pl.pallas_call(
        _final_body,
        grid=(_NP // _BR,),
        in_specs=[
            pl.BlockSpec((2, _BR, 128), lambda i: (0, i, 0)),
            pl.BlockSpec((2, _BR, 128), lambda i: (0, i, 0)),
            pl.BlockSpec((_BR, 128), lambda i: (i, 0)),
            pl.BlockSpec((1, _F), lambda i: (0, 0)),
            pl.BlockSpec((_BR, _F), lambda i: (i, 0)),
            pl.BlockSpec((_F, _F), lambda i: (0, 0)),
            pl.BlockSpec((1, _F), lambda i: (0, 0)),
            pl.BlockSpec((_F, _F), lambda i: (0, 0)),
            pl.BlockSpec((1, _F), lambda i: (0, 0)),
            pl.BlockSpec((_F, 128), lambda i: (0, 0)),
            pl.BlockSpec((1, 128), lambda i: (0, 0)),
        ],
        out_specs=pl.BlockSpec((_BR, 128), lambda i: (i, 0)),
        out_shape=jax.ShapeDtypeStruct((_NP, 128), jnp.float32),
    )(acc, ztp, dinvb, b2d, y1, W1, b1, W2, b2, W3p, b3p)


def kernel(x, edge_index, TRAIN, Wc0, bc0, Wc1, bc1, Wc2, bc2,
           Wf1, bf1, Wf2, bf2, Wf3, bf3):
    pad = jnp.full((_EP - _E,), _N, jnp.int32)
    src3 = jnp.concatenate([edge_index[0], pad]).reshape(16, _NCH, _EC)
    dst3 = jnp.concatenate([edge_index[1], pad]).reshape(16, _NCH, _EC)
    xp = jnp.pad(x, ((0, _NP - _N), (0, 0)))

    deg2 = _deg_kernel(dst3)
    zt0, dinvb = _pre(xp, Wc0, deg2)
    acc1 = _conv_kernel(zt0, src3, dst3)
    zt1, y1 = _mid(acc1, zt0, dinvb, bc0.reshape(1, -1), Wc1)
    acc2 = _conv_kernel(zt1, src3, dst3)
    zt2, _ = _mid(acc2, zt1, dinvb, bc1.reshape(1, -1), Wc2, res=y1)
    acc3 = _conv_kernel(zt2, src3, dst3)
    outp = _final(acc3, zt2, dinvb, bc2.reshape(1, -1), y1,
                  Wf1, bf1.reshape(1, -1), Wf2, bf2.reshape(1, -1),
                  jnp.pad(Wf3, ((0, 0), (0, 128 - 7))),
                  jnp.pad(bf3, (0, 128 - 7)).reshape(1, 128))
    return outp[:_N, :7]


# speedup vs baseline: 1.0419x; 1.0419x over previous
import functools

import jax
import jax.numpy as jnp
from jax import lax
from jax.experimental import pallas as pl
from jax.experimental.pallas import tpu as pltpu
from jax.experimental.pallas import tpu_sc as plsc

_N = 10000
_E = 160000
_F = 256
_NP = 10240
_EP = 163840
_CH = 128
_EC = 64
_NCH = _EP // 16 // _EC
_WCH = 32
_BR = 512

_mesh = plsc.VectorSubcoreMesh(core_axis_name="c", subcore_axis_name="s")


@functools.partial(
    pl.kernel,
    out_type=jax.ShapeDtypeStruct((2, _NP), jnp.float32),
    mesh=_mesh,
    scratch_types=[
        pltpu.VMEM((80, _EC), jnp.int32),
        pltpu.VMEM((_CH,), jnp.float32),
        pltpu.VMEM((_CH,), jnp.float32),
        pltpu.VMEM_SHARED((_NP,), jnp.float32),
    ],
)
def _deg_kernel(dst_hbm, out_hbm, idx_v, ones_v, zero_v, acc):
    cid = lax.axis_index("c")
    sid = lax.axis_index("s")
    for k in range(8):
        ones_v[pl.ds(k * 16, 16)] = jnp.ones((16,), jnp.float32)
        zero_v[pl.ds(k * 16, 16)] = jnp.zeros((16,), jnp.float32)
    base = sid * (_NP // 16)
    for j in range(_NP // 16 // _CH):
        pltpu.sync_copy(zero_v, acc.at[pl.ds(base + j * _CH, _CH)])
    plsc.subcore_barrier()
    pltpu.sync_copy(dst_hbm.at[sid, pl.ds(cid * 80, 80)], idx_v)

    @pl.loop(0, 80)
    def _(g):
        pltpu.sync_copy(ones_v.at[pl.ds(0, _EC)], acc.at[idx_v.at[g]],
                        add=True)

    plsc.subcore_barrier()
    pltpu.sync_copy(acc.at[pl.ds(base, _NP // 16)],
                    out_hbm.at[cid, pl.ds(base, _NP // 16)])


@functools.partial(
    pl.kernel,
    out_type=jax.ShapeDtypeStruct((2, _NP, 128), jnp.float32),
    mesh=_mesh,
    scratch_types=[
        pltpu.VMEM((_WCH, _EC), jnp.int32),
        pltpu.VMEM((_WCH, _EC), jnp.int32),
        pltpu.VMEM((_EC, 128), jnp.float32),
        pltpu.VMEM((_EC, 128), jnp.float32),
        pltpu.VMEM((_EC, 128), jnp.float32),
        pltpu.VMEM((_EC, 128), jnp.float32),
        pltpu.VMEM_SHARED((_NP, 128), jnp.float32),
        pltpu.SemaphoreType.DMA,
        pltpu.SemaphoreType.DMA,
        pltpu.SemaphoreType.DMA,
        pltpu.SemaphoreType.DMA,
    ],
)
def _conv_kernel(tbl_hbm, src_hbm, dst_hbm, out_hbm, src_w, dst_w, rb0, rb1,
                 rb2, rb3, acc, sm0, sm1, sm2, sm3):
    cid = lax.axis_index("c")
    sid = lax.axis_index("s")
    rows_v = (rb0, rb1, rb2, rb3)
    sems = (sm0, sm1, sm2, sm3)
    rows_per = _NP // 16
    base = sid * rows_per

    @pl.loop(0, _EC)
    def _(i):
        for k in range(8):
            rows_v[0][i, pl.ds(k * 16, 16)] = jnp.zeros((16,), jnp.float32)

    for j in range(rows_per // _EC):
        pltpu.sync_copy(rows_v[0], acc.at[pl.ds(base + j * _EC, _EC)])
    plsc.subcore_barrier()

    tbl = tbl_hbm.at[cid]

    @pl.loop(0, _NCH // _WCH)
    def _(w):
        pltpu.sync_copy(src_hbm.at[sid, pl.ds(w * _WCH, _WCH)], src_w)
        pltpu.sync_copy(dst_hbm.at[sid, pl.ds(w * _WCH, _WCH)], dst_w)

        for i in range(4):
            pltpu.async_copy(tbl.at[src_w.at[i]], rows_v[i], sems[i])

        @pl.loop(0, _WCH, step=4)
        def _(g):
            for i in range(4):
                pltpu.make_async_copy(tbl.at[src_w.at[g + i]], rows_v[i],
                                      sems[i]).wait()
                pltpu.sync_copy(rows_v[i], acc.at[dst_w.at[g + i]], add=True)
                gnext = jnp.minimum(g + 4 + i, _WCH - 1)
                pltpu.async_copy(tbl.at[src_w.at[gnext]], rows_v[i], sems[i])

        for i in range(4):
            pltpu.make_async_copy(tbl.at[src_w.at[_WCH - 1]], rows_v[i],
                                  sems[i]).wait()

    plsc.subcore_barrier()
    pltpu.sync_copy(acc.at[pl.ds(base, rows_per)],
                    out_hbm.at[cid, pl.ds(base, rows_per)])


def _colvec(row, n):
    eye = (lax.broadcasted_iota(jnp.int32, (n, n), 0)
           == lax.broadcasted_iota(jnp.int32, (n, n), 1)).astype(jnp.float32)
    return lax.dot_general(eye, row, (((1,), (1,)), ((), ())),
                           preferred_element_type=jnp.float32)


def _pre_body(x_ref, w_ref, deg_ref, zt_ref, dinv_ref):
    i = pl.program_id(0)
    y = jnp.dot(x_ref[...], w_ref[...], preferred_element_type=jnp.float32)
    dsl = deg_ref[:, pl.ds(i * _BR, _BR)]
    dl = lax.rsqrt(dsl[0:1] + dsl[1:2] + 1.0)
    dc = _colvec(dl, _BR)
    z = y * dc
    zt_ref[0] = z[:, :128]
    zt_ref[1] = z[:, 128:]
    dinv_ref[...] = jnp.broadcast_to(dc, (_BR, 128))


def _pre(xp, W, deg2):
    return pl.pallas_call(
        _pre_body,
        grid=(_NP // _BR,),
        in_specs=[
            pl.BlockSpec((_BR, _F), lambda i: (i, 0)),
            pl.BlockSpec((_F, _F), lambda i: (0, 0)),
            pl.BlockSpec((2, _NP), lambda i: (0, 0)),
        ],
        out_specs=[
            pl.BlockSpec((2, _BR, 128), lambda i: (0, i, 0)),
            pl.BlockSpec((_BR, 128), lambda i: (i, 0)),
        ],
        out_shape=[
            jax.ShapeDtypeStruct((2, _NP, 128), jnp.float32),
            jax.ShapeDtypeStruct((_NP, 128), jnp.float32),
        ],
    )(xp, W, deg2)


def _mid_body(res, acc_ref, ztp_ref, dinv_ref, b_ref, w_ref, *rest):
    if res:
        res_ref, zt_ref, y_ref = rest
    else:
        zt_ref, y_ref = rest
    dinv = dinv_ref[...]
    h0 = (acc_ref[0] + ztp_ref[0]) * dinv
    h1 = (acc_ref[1] + ztp_ref[1]) * dinv
    y = jnp.concatenate([h0, h1], axis=1) + b_ref[...]
    if res:
        y = y + res_ref[...]
    y = jnp.maximum(y, 0.0)
    y_ref[...] = y
    z = jnp.dot(y, w_ref[...], preferred_element_type=jnp.float32)
    zt_ref[0] = z[:, :128] * dinv
    zt_ref[1] = z[:, 128:] * dinv


def _mid(acc, ztp, dinvb, b2, W, res=None):
    in_specs = [
        pl.BlockSpec((2, _BR, 128), lambda i: (0, i, 0)),
        pl.BlockSpec((2, _BR, 128), lambda i: (0, i, 0)),
        pl.BlockSpec((_BR, 128), lambda i: (i, 0)),
        pl.BlockSpec((1, _F), lambda i: (0, 0)),
        pl.BlockSpec((_F, _F), lambda i: (0, 0)),
    ]
    args = [acc, ztp, dinvb, b2, W]
    if res is not None:
        in_specs.append(pl.BlockSpec((_BR, _F), lambda i: (i, 0)))
        args.append(res)
    return pl.pallas_call(
        functools.partial(_mid_body, res is not None),
        grid=(_NP // _BR,),
        in_specs=in_specs,
        out_specs=[
            pl.BlockSpec((2, _BR, 128), lambda i: (0, i, 0)),
            pl.BlockSpec((_BR, _F), lambda i: (i, 0)),
        ],
        out_shape=[
            jax.ShapeDtypeStruct((2, _NP, 128), jnp.float32),
            jax.ShapeDtypeStruct((_NP, _F), jnp.float32),
        ],
    )(*args)


def _final_body(acc_ref, ztp_ref, dinv_ref, b_ref, res_ref, w1_ref, b1_ref,
                w2_ref, b2_ref, w3_ref, b3_ref, out_ref):
    dinv = dinv_ref[...]
    h0 = (acc_ref[0] + ztp_ref[0]) * dinv
    h1 = (acc_ref[1] + ztp_ref[1]) * dinv
    y = jnp.concatenate([h0, h1], axis=1) + b_ref[...] + res_ref[...]
    y = jnp.maximum(y, 0.0)
    y = jnp.maximum(jnp.dot(y, w1_ref[...],
                            preferred_element_type=jnp.float32) + b1_ref[...],
                    0.0)
    y = jnp.maximum(jnp.dot(y, w2_ref[...],
                            preferred_element_type=jnp.float32) + b2_ref[...],
                    0.0)
    p = jnp.dot(y, w3_ref[...], preferred_element_type=jnp.float32) + b3_ref[...]
    col = lax.broadcasted_iota(jnp.int32, (_BR, 128), 1)
    pm = jnp.where(col < 7, p, -1e30)
    m = jnp.max(pm, axis=1, keepdims=True)
    s = jnp.sum(jnp.exp(pm - m), axis=1, keepdims=True)
    out_ref[...] = p - m - jnp.log(s)


def _final(acc, ztp, dinvb, b2d, y1, W1, b1, W2, b2, W3p, b3p):
    return pl.pallas_call(
        _final_body,
        grid=(_NP // _BR,),
        in_specs=[
            pl.BlockSpec((2, _BR, 128), lambda i: (0, i, 0)),
            pl.BlockSpec((2, _BR, 128), lambda i: (0, i, 0)),
            pl.BlockSpec((_BR, 128), lambda i: (i, 0)),
            pl.BlockSpec((1, _F), lambda i: (0, 0)),
            pl.BlockSpec((_BR, _F), lambda i: (i, 0)),
            pl.BlockSpec((_F, _F), lambda i: (0, 0)),
            pl.BlockSpec((1, _F), lambda i: (0, 0)),
            pl.BlockSpec((_F, _F), lambda i: (0, 0)),
            pl.BlockSpec((1, _F), lambda i: (0, 0)),
            pl.BlockSpec((_F, 128), lambda i: (0, 0)),
            pl.BlockSpec((1, 128), lambda i: (0, 0)),
        ],
        out_specs=pl.BlockSpec((_BR, 128), lambda i: (i, 0)),
        out_shape=jax.ShapeDtypeStruct((_NP, 128), jnp.float32),
    )(acc, ztp, dinvb, b2d, y1, W1, b1, W2, b2, W3p, b3p)


def kernel(x, edge_index, TRAIN, Wc0, bc0, Wc1, bc1, Wc2, bc2,
           Wf1, bf1, Wf2, bf2, Wf3, bf3):
    pad = jnp.full((_EP - _E,), _N, jnp.int32)
    src3 = jnp.concatenate([edge_index[0], pad]).reshape(16, _NCH, _EC)
    dst3 = jnp.concatenate([edge_index[1], pad]).reshape(16, _NCH, _EC)
    xp = jnp.pad(x, ((0, _NP - _N), (0, 0)))

    deg2 = _deg_kernel(dst3)
    zt0, dinvb = _pre(xp, Wc0, deg2)
    acc1 = _conv_kernel(zt0, src3, dst3)
    zt1, y1 = _mid(acc1, zt0, dinvb, bc0.reshape(1, -1), Wc1)
    acc2 = _conv_kernel(zt1, src3, dst3)
    zt2, _ = _mid(acc2, zt1, dinvb, bc1.reshape(1, -1), Wc2, res=y1)
    acc3 = _conv_kernel(zt2, src3, dst3)
    outp = _final(acc3, zt2, dinvb, bc2.reshape(1, -1), y1,
                  Wf1, bf1.reshape(1, -1), Wf2, bf2.reshape(1, -1),
                  jnp.pad(Wf3, ((0, 0), (0, 128 - 7))),
                  jnp.pad(bf3, (0, 128 - 7)).reshape(1, 128))
    return outp[:_N, :7]


# speedup vs baseline: 1.0637x; 1.0209x over previous
import functools

import jax
import jax.numpy as jnp
from jax import lax
from jax.experimental import pallas as pl
from jax.experimental.pallas import tpu as pltpu
from jax.experimental.pallas import tpu_sc as plsc

_N = 10000
_E = 160000
_F = 256
_NP = 10240
_EP = 163840
_CH = 128
_EC = 64
_NCH = _EP // 16 // _EC
_WCH = 32
_BR = 1024

_mesh = plsc.VectorSubcoreMesh(core_axis_name="c", subcore_axis_name="s")


@functools.partial(
    pl.kernel,
    out_type=jax.ShapeDtypeStruct((2, _NP), jnp.float32),
    mesh=_mesh,
    scratch_types=[
        pltpu.VMEM((80, _EC), jnp.int32),
        pltpu.VMEM((_CH,), jnp.float32),
        pltpu.VMEM((_CH,), jnp.float32),
        pltpu.VMEM_SHARED((_NP,), jnp.float32),
    ],
)
def _deg_kernel(dst_hbm, out_hbm, idx_v, ones_v, zero_v, acc):
    cid = lax.axis_index("c")
    sid = lax.axis_index("s")
    for k in range(8):
        ones_v[pl.ds(k * 16, 16)] = jnp.ones((16,), jnp.float32)
        zero_v[pl.ds(k * 16, 16)] = jnp.zeros((16,), jnp.float32)
    base = sid * (_NP // 16)
    for j in range(_NP // 16 // _CH):
        pltpu.sync_copy(zero_v, acc.at[pl.ds(base + j * _CH, _CH)])
    plsc.subcore_barrier()
    pltpu.sync_copy(dst_hbm.at[sid, pl.ds(cid * 80, 80)], idx_v)

    @pl.loop(0, 80)
    def _(g):
        pltpu.sync_copy(ones_v.at[pl.ds(0, _EC)], acc.at[idx_v.at[g]],
                        add=True)

    plsc.subcore_barrier()
    pltpu.sync_copy(acc.at[pl.ds(base, _NP // 16)],
                    out_hbm.at[cid, pl.ds(base, _NP // 16)])


@functools.partial(
    pl.kernel,
    out_type=jax.ShapeDtypeStruct((2, _NP, 128), jnp.float32),
    mesh=_mesh,
    scratch_types=[
        pltpu.VMEM((_WCH, _EC), jnp.int32),
        pltpu.VMEM((_WCH, _EC), jnp.int32),
        pltpu.VMEM((_EC, 128), jnp.float32),
        pltpu.VMEM((_EC, 128), jnp.float32),
        pltpu.VMEM((_EC, 128), jnp.float32),
        pltpu.VMEM((_EC, 128), jnp.float32),
        pltpu.VMEM_SHARED((_NP, 128), jnp.float32),
        pltpu.SemaphoreType.DMA,
        pltpu.SemaphoreType.DMA,
        pltpu.SemaphoreType.DMA,
        pltpu.SemaphoreType.DMA,
    ],
)
def _conv_kernel(tbl_hbm, src_hbm, dst_hbm, out_hbm, src_w, dst_w, rb0, rb1,
                 rb2, rb3, acc, sm0, sm1, sm2, sm3):
    cid = lax.axis_index("c")
    sid = lax.axis_index("s")
    rows_v = (rb0, rb1, rb2, rb3)
    sems = (sm0, sm1, sm2, sm3)
    rows_per = _NP // 16
    base = sid * rows_per

    @pl.loop(0, _EC)
    def _(i):
        for k in range(8):
            rows_v[0][i, pl.ds(k * 16, 16)] = jnp.zeros((16,), jnp.float32)

    for j in range(rows_per // _EC):
        pltpu.sync_copy(rows_v[0], acc.at[pl.ds(base + j * _EC, _EC)])
    plsc.subcore_barrier()

    tbl = tbl_hbm.at[cid]

    @pl.loop(0, _NCH // _WCH)
    def _(w):
        pltpu.sync_copy(src_hbm.at[sid, pl.ds(w * _WCH, _WCH)], src_w)
        pltpu.sync_copy(dst_hbm.at[sid, pl.ds(w * _WCH, _WCH)], dst_w)

        for i in range(4):
            pltpu.async_copy(tbl.at[src_w.at[i]], rows_v[i], sems[i])

        @pl.loop(0, _WCH, step=4)
        def _(g):
            for i in range(4):
                pltpu.make_async_copy(tbl.at[src_w.at[g + i]], rows_v[i],
                                      sems[i]).wait()
                pltpu.sync_copy(rows_v[i], acc.at[dst_w.at[g + i]], add=True)
                gnext = jnp.minimum(g + 4 + i, _WCH - 1)
                pltpu.async_copy(tbl.at[src_w.at[gnext]], rows_v[i], sems[i])

        for i in range(4):
            pltpu.make_async_copy(tbl.at[src_w.at[_WCH - 1]], rows_v[i],
                                  sems[i]).wait()

    plsc.subcore_barrier()
    pltpu.sync_copy(acc.at[pl.ds(base, rows_per)],
                    out_hbm.at[cid, pl.ds(base, rows_per)])


def _colvec(row, n):
    eye = (lax.broadcasted_iota(jnp.int32, (n, n), 0)
           == lax.broadcasted_iota(jnp.int32, (n, n), 1)).astype(jnp.float32)
    return lax.dot_general(eye, row, (((1,), (1,)), ((), ())),
                           preferred_element_type=jnp.float32)


def _pre_body(x_ref, w_ref, deg_ref, zt_ref, dinv_ref):
    i = pl.program_id(0)
    y = jnp.dot(x_ref[...], w_ref[...], preferred_element_type=jnp.float32)
    dsl = deg_ref[:, pl.ds(i * _BR, _BR)]
    dl = lax.rsqrt(dsl[0:1] + dsl[1:2] + 1.0)
    dc = _colvec(dl, _BR)
    z = y * dc
    zt_ref[0] = z[:, :128]
    zt_ref[1] = z[:, 128:]
    dinv_ref[...] = jnp.broadcast_to(dc, (_BR, 128))


def _pre(xp, W, deg2):
    return pl.pallas_call(
        _pre_body,
        grid=(_NP // _BR,),
        in_specs=[
            pl.BlockSpec((_BR, _F), lambda i: (i, 0)),
            pl.BlockSpec((_F, _F), lambda i: (0, 0)),
            pl.BlockSpec((2, _NP), lambda i: (0, 0)),
        ],
        out_specs=[
            pl.BlockSpec((2, _BR, 128), lambda i: (0, i, 0)),
            pl.BlockSpec((_BR, 128), lambda i: (i, 0)),
        ],
        out_shape=[
            jax.ShapeDtypeStruct((2, _NP, 128), jnp.float32),
            jax.ShapeDtypeStruct((_NP, 128), jnp.float32),
        ],
    )(xp, W, deg2)


def _mid_body(res, acc_ref, ztp_ref, dinv_ref, b_ref, w_ref, *rest):
    if res:
        res_ref, zt_ref, y_ref = rest
    else:
        zt_ref, y_ref = rest
    dinv = dinv_ref[...]
    h0 = (acc_ref[0] + ztp_ref[0]) * dinv
    h1 = (acc_ref[1] + ztp_ref[1]) * dinv
    y = jnp.concatenate([h0, h1], axis=1) + b_ref[...]
    if res:
        y = y + res_ref[...]
    y = jnp.maximum(y, 0.0)
    y_ref[...] = y
    z = jnp.dot(y, w_ref[...], preferred_element_type=jnp.float32)
    zt_ref[0] = z[:, :128] * dinv
    zt_ref[1] = z[:, 128:] * dinv


def _mid(acc, ztp, dinvb, b2, W, res=None):
    in_specs = [
        pl.BlockSpec((2, _BR, 128), lambda i: (0, i, 0)),
        pl.BlockSpec((2, _BR, 128), lambda i: (0, i, 0)),
        pl.BlockSpec((_BR, 128), lambda i: (i, 0)),
        pl.BlockSpec((1, _F), lambda i: (0, 0)),
        pl.BlockSpec((_F, _F), lambda i: (0, 0)),
    ]
    args = [acc, ztp, dinvb, b2, W]
    if res is not None:
        in_specs.append(pl.BlockSpec((_BR, _F), lambda i: (i, 0)))
        args.append(res)
    return pl.pallas_call(
        functools.partial(_mid_body, res is not None),
        grid=(_NP // _BR,),
        in_specs=in_specs,
        out_specs=[
            pl.BlockSpec((2, _BR, 128), lambda i: (0, i, 0)),
            pl.BlockSpec((_BR, _F), lambda i: (i, 0)),
        ],
        out_shape=[
            jax.ShapeDtypeStruct((2, _NP, 128), jnp.float32),
            jax.ShapeDtypeStruct((_NP, _F), jnp.float32),
        ],
    )(*args)


def _final_body(acc_ref, ztp_ref, dinv_ref, b_ref, res_ref, w1_ref, b1_ref,
                w2_ref, b2_ref, w3_ref, b3_ref, out_ref):
    dinv = dinv_ref[...]
    h0 = (acc_ref[0] + ztp_ref[0]) * dinv
    h1 = (acc_ref[1] + ztp_ref[1]) * dinv
    y = jnp.concatenate([h0, h1], axis=1) + b_ref[...] + res_ref[...]
    y = jnp.maximum(y, 0.0)
    y = jnp.maximum(jnp.dot(y, w1_ref[...],
                            preferred_element_type=jnp.float32) + b1_ref[...],
                    0.0)
    y = jnp.maximum(jnp.dot(y, w2_ref[...],
                            preferred_element_type=jnp.float32) + b2_ref[...],
                    0.0)
    p = jnp.dot(y, w3_ref[...], preferred_element_type=jnp.float32) + b3_ref[...]
    col = lax.broadcasted_iota(jnp.int32, (_BR, 128), 1)
    pm = jnp.where(col < 7, p, -1e30)
    m = jnp.max(pm, axis=1, keepdims=True)
    s = jnp.sum(jnp.exp(pm - m), axis=1, keepdims=True)
    out_ref[...] = p - m - jnp.log(s)


def _final(acc, ztp, dinvb, b2d, y1, W1, b1, W2, b2, W3p, b3p):
    return pl.pallas_call(
        _final_body,
        grid=(_NP // _BR,),
        in_specs=[
            pl.BlockSpec((2, _BR, 128), lambda i: (0, i, 0)),
            pl.BlockSpec((2, _BR, 128), lambda i: (0, i, 0)),
            pl.BlockSpec((_BR, 128), lambda i: (i, 0)),
            pl.BlockSpec((1, _F), lambda i: (0, 0)),
            pl.BlockSpec((_BR, _F), lambda i: (i, 0)),
            pl.BlockSpec((_F, _F), lambda i: (0, 0)),
            pl.BlockSpec((1, _F), lambda i: (0, 0)),
            pl.BlockSpec((_F, _F), lambda i: (0, 0)),
            pl.BlockSpec((1, _F), lambda i: (0, 0)),
            pl.BlockSpec((_F, 128), lambda i: (0, 0)),
            pl.BlockSpec((1, 128), lambda i: (0, 0)),
        ],
        out_specs=pl.BlockSpec((_BR, 128), lambda i: (i, 0)),
        out_shape=jax.ShapeDtypeStruct((_NP, 128), jnp.float32),
    )(acc, ztp, dinvb, b2d, y1, W1, b1, W2, b2, W3p, b3p)


def kernel(x, edge_index, TRAIN, Wc0, bc0, Wc1, bc1, Wc2, bc2,
           Wf1, bf1, Wf2, bf2, Wf3, bf3):
    pad = jnp.full((_EP - _E,), _N, jnp.int32)
    src3 = jnp.concatenate([edge_index[0], pad]).reshape(16, _NCH, _EC)
    dst3 = jnp.concatenate([edge_index[1], pad]).reshape(16, _NCH, _EC)
    xp = jnp.pad(x, ((0, _NP - _N), (0, 0)))

    deg2 = _deg_kernel(dst3)
    zt0, dinvb = _pre(xp, Wc0, deg2)
    acc1 = _conv_kernel(zt0, src3, dst3)
    zt1, y1 = _mid(acc1, zt0, dinvb, bc0.reshape(1, -1), Wc1)
    acc2 = _conv_kernel(zt1, src3, dst3)
    zt2, _ = _mid(acc2, zt1, dinvb, bc1.reshape(1, -1), Wc2, res=y1)
    acc3 = _conv_kernel(zt2, src3, dst3)
    outp = _final(acc3, zt2, dinvb, bc2.reshape(1, -1), y1,
                  Wf1, bf1.reshape(1, -1), Wf2, bf2.reshape(1, -1),
                  jnp.pad(Wf3, ((0, 0), (0, 128 - 7))),
                  jnp.pad(bf3, (0, 128 - 7)).reshape(1, 128))
    return outp[:_N, :7]


# speedup vs baseline: 1.0710x; 1.0069x over previous
import functools

import jax
import jax.numpy as jnp
from jax import lax
from jax.experimental import pallas as pl
from jax.experimental.pallas import tpu as pltpu
from jax.experimental.pallas import tpu_sc as plsc

_N = 10000
_E = 160000
_F = 256
_NP = 10240
_EP = 163840
_CH = 128
_EC = 64
_NCH = _EP // 16 // _EC
_WCH = 32
_BR = 2048

_mesh = plsc.VectorSubcoreMesh(core_axis_name="c", subcore_axis_name="s")


@functools.partial(
    pl.kernel,
    out_type=jax.ShapeDtypeStruct((2, _NP), jnp.float32),
    mesh=_mesh,
    scratch_types=[
        pltpu.VMEM((80, _EC), jnp.int32),
        pltpu.VMEM((_CH,), jnp.float32),
        pltpu.VMEM((_CH,), jnp.float32),
        pltpu.VMEM_SHARED((_NP,), jnp.float32),
    ],
)
def _deg_kernel(dst_hbm, out_hbm, idx_v, ones_v, zero_v, acc):
    cid = lax.axis_index("c")
    sid = lax.axis_index("s")
    for k in range(8):
        ones_v[pl.ds(k * 16, 16)] = jnp.ones((16,), jnp.float32)
        zero_v[pl.ds(k * 16, 16)] = jnp.zeros((16,), jnp.float32)
    base = sid * (_NP // 16)
    for j in range(_NP // 16 // _CH):
        pltpu.sync_copy(zero_v, acc.at[pl.ds(base + j * _CH, _CH)])
    plsc.subcore_barrier()
    pltpu.sync_copy(dst_hbm.at[sid, pl.ds(cid * 80, 80)], idx_v)

    @pl.loop(0, 80)
    def _(g):
        pltpu.sync_copy(ones_v.at[pl.ds(0, _EC)], acc.at[idx_v.at[g]],
                        add=True)

    plsc.subcore_barrier()
    pltpu.sync_copy(acc.at[pl.ds(base, _NP // 16)],
                    out_hbm.at[cid, pl.ds(base, _NP // 16)])


@functools.partial(
    pl.kernel,
    out_type=jax.ShapeDtypeStruct((2, _NP, 128), jnp.float32),
    mesh=_mesh,
    scratch_types=[
        pltpu.VMEM((_WCH, _EC), jnp.int32),
        pltpu.VMEM((_WCH, _EC), jnp.int32),
        pltpu.VMEM((_EC, 128), jnp.float32),
        pltpu.VMEM((_EC, 128), jnp.float32),
        pltpu.VMEM((_EC, 128), jnp.float32),
        pltpu.VMEM((_EC, 128), jnp.float32),
        pltpu.VMEM_SHARED((_NP, 128), jnp.float32),
        pltpu.SemaphoreType.DMA,
        pltpu.SemaphoreType.DMA,
        pltpu.SemaphoreType.DMA,
        pltpu.SemaphoreType.DMA,
    ],
)
def _conv_kernel(tbl_hbm, src_hbm, dst_hbm, out_hbm, src_w, dst_w, rb0, rb1,
                 rb2, rb3, acc, sm0, sm1, sm2, sm3):
    cid = lax.axis_index("c")
    sid = lax.axis_index("s")
    rows_v = (rb0, rb1, rb2, rb3)
    sems = (sm0, sm1, sm2, sm3)
    rows_per = _NP // 16
    base = sid * rows_per

    @pl.loop(0, _EC)
    def _(i):
        for k in range(8):
            rows_v[0][i, pl.ds(k * 16, 16)] = jnp.zeros((16,), jnp.float32)

    for j in range(rows_per // _EC):
        pltpu.sync_copy(rows_v[0], acc.at[pl.ds(base + j * _EC, _EC)])
    plsc.subcore_barrier()

    tbl = tbl_hbm.at[cid]

    @pl.loop(0, _NCH // _WCH)
    def _(w):
        pltpu.sync_copy(src_hbm.at[sid, pl.ds(w * _WCH, _WCH)], src_w)
        pltpu.sync_copy(dst_hbm.at[sid, pl.ds(w * _WCH, _WCH)], dst_w)

        for i in range(4):
            pltpu.async_copy(tbl.at[src_w.at[i]], rows_v[i], sems[i])

        @pl.loop(0, _WCH, step=4)
        def _(g):
            for i in range(4):
                pltpu.make_async_copy(tbl.at[src_w.at[g + i]], rows_v[i],
                                      sems[i]).wait()
                pltpu.sync_copy(rows_v[i], acc.at[dst_w.at[g + i]], add=True)
                gnext = jnp.minimum(g + 4 + i, _WCH - 1)
                pltpu.async_copy(tbl.at[src_w.at[gnext]], rows_v[i], sems[i])

        for i in range(4):
            pltpu.make_async_copy(tbl.at[src_w.at[_WCH - 1]], rows_v[i],
                                  sems[i]).wait()

    plsc.subcore_barrier()
    pltpu.sync_copy(acc.at[pl.ds(base, rows_per)],
                    out_hbm.at[cid, pl.ds(base, rows_per)])


def _colvec(row, n):
    eye = (lax.broadcasted_iota(jnp.int32, (n, n), 0)
           == lax.broadcasted_iota(jnp.int32, (n, n), 1)).astype(jnp.float32)
    return lax.dot_general(eye, row, (((1,), (1,)), ((), ())),
                           preferred_element_type=jnp.float32)


def _pre_body(x_ref, w_ref, deg_ref, zt_ref, dinv_ref):
    i = pl.program_id(0)
    y = jnp.dot(x_ref[...], w_ref[...], preferred_element_type=jnp.float32)
    dsl = deg_ref[:, pl.ds(i * _BR, _BR)]
    dl = lax.rsqrt(dsl[0:1] + dsl[1:2] + 1.0)
    dc = _colvec(dl, _BR)
    z = y * dc
    zt_ref[0] = z[:, :128]
    zt_ref[1] = z[:, 128:]
    dinv_ref[...] = jnp.broadcast_to(dc, (_BR, 128))


def _pre(xp, W, deg2):
    return pl.pallas_call(
        _pre_body,
        grid=(_NP // _BR,),
        in_specs=[
            pl.BlockSpec((_BR, _F), lambda i: (i, 0)),
            pl.BlockSpec((_F, _F), lambda i: (0, 0)),
            pl.BlockSpec((2, _NP), lambda i: (0, 0)),
        ],
        out_specs=[
            pl.BlockSpec((2, _BR, 128), lambda i: (0, i, 0)),
            pl.BlockSpec((_BR, 128), lambda i: (i, 0)),
        ],
        out_shape=[
            jax.ShapeDtypeStruct((2, _NP, 128), jnp.float32),
            jax.ShapeDtypeStruct((_NP, 128), jnp.float32),
        ],
    )(xp, W, deg2)


def _mid_body(res, acc_ref, ztp_ref, dinv_ref, b_ref, w_ref, *rest):
    if res:
        res_ref, zt_ref, y_ref = rest
    else:
        zt_ref, y_ref = rest
    dinv = dinv_ref[...]
    h0 = (acc_ref[0] + ztp_ref[0]) * dinv
    h1 = (acc_ref[1] + ztp_ref[1]) * dinv
    y = jnp.concatenate([h0, h1], axis=1) + b_ref[...]
    if res:
        y = y + res_ref[...]
    y = jnp.maximum(y, 0.0)
    y_ref[...] = y
    z = jnp.dot(y, w_ref[...], preferred_element_type=jnp.float32)
    zt_ref[0] = z[:, :128] * dinv
    zt_ref[1] = z[:, 128:] * dinv


def _mid(acc, ztp, dinvb, b2, W, res=None):
    in_specs = [
        pl.BlockSpec((2, _BR, 128), lambda i: (0, i, 0)),
        pl.BlockSpec((2, _BR, 128), lambda i: (0, i, 0)),
        pl.BlockSpec((_BR, 128), lambda i: (i, 0)),
        pl.BlockSpec((1, _F), lambda i: (0, 0)),
        pl.BlockSpec((_F, _F), lambda i: (0, 0)),
    ]
    args = [acc, ztp, dinvb, b2, W]
    if res is not None:
        in_specs.append(pl.BlockSpec((_BR, _F), lambda i: (i, 0)))
        args.append(res)
    return pl.pallas_call(
        functools.partial(_mid_body, res is not None),
        grid=(_NP // _BR,),
        in_specs=in_specs,
        out_specs=[
            pl.BlockSpec((2, _BR, 128), lambda i: (0, i, 0)),
            pl.BlockSpec((_BR, _F), lambda i: (i, 0)),
        ],
        out_shape=[
            jax.ShapeDtypeStruct((2, _NP, 128), jnp.float32),
            jax.ShapeDtypeStruct((_NP, _F), jnp.float32),
        ],
    )(*args)


def _final_body(acc_ref, ztp_ref, dinv_ref, b_ref, res_ref, w1_ref, b1_ref,
                w2_ref, b2_ref, w3_ref, b3_ref, out_ref):
    dinv = dinv_ref[...]
    h0 = (acc_ref[0] + ztp_ref[0]) * dinv
    h1 = (acc_ref[1] + ztp_ref[1]) * dinv
    y = jnp.concatenate([h0, h1], axis=1) + b_ref[...] + res_ref[...]
    y = jnp.maximum(y, 0.0)
    y = jnp.maximum(jnp.dot(y, w1_ref[...],
                            preferred_element_type=jnp.float32) + b1_ref[...],
                    0.0)
    y = jnp.maximum(jnp.dot(y, w2_ref[...],
                            preferred_element_type=jnp.float32) + b2_ref[...],
                    0.0)
    p = jnp.dot(y, w3_ref[...], preferred_element_type=jnp.float32) + b3_ref[...]
    col = lax.broadcasted_iota(jnp.int32, (_BR, 128), 1)
    pm = jnp.where(col < 7, p, -1e30)
    m = jnp.max(pm, axis=1, keepdims=True)
    s = jnp.sum(jnp.exp(pm - m), axis=1, keepdims=True)
    out_ref[...] = p - m - jnp.log(s)


def _final(acc, ztp, dinvb, b2d, y1, W1, b1, W2, b2, W3p, b3p):
    return pl.pallas_call(
        _final_body,
        grid=(_NP // _BR,),
        in_specs=[
            pl.BlockSpec((2, _BR, 128), lambda i: (0, i, 0)),
            pl.BlockSpec((2, _BR, 128), lambda i: (0, i, 0)),
            pl.BlockSpec((_BR, 128), lambda i: (i, 0)),
            pl.BlockSpec((1, _F), lambda i: (0, 0)),
            pl.BlockSpec((_BR, _F), lambda i: (i, 0)),
            pl.BlockSpec((_F, _F), lambda i: (0, 0)),
            pl.BlockSpec((1, _F), lambda i: (0, 0)),
            pl.BlockSpec((_F, _F), lambda i: (0, 0)),
            pl.BlockSpec((1, _F), lambda i: (0, 0)),
            pl.BlockSpec((_F, 128), lambda i: (0, 0)),
            pl.BlockSpec((1, 128), lambda i: (0, 0)),
        ],
        out_specs=pl.BlockSpec((_BR, 128), lambda i: (i, 0)),
        out_shape=jax.ShapeDtypeStruct((_NP, 128), jnp.float32),
    )(acc, ztp, dinvb, b2d, y1, W1, b1, W2, b2, W3p, b3p)


def kernel(x, edge_index, TRAIN, Wc0, bc0, Wc1, bc1, Wc2, bc2,
           Wf1, bf1, Wf2, bf2, Wf3, bf3):
    pad = jnp.full((_EP - _E,), _N, jnp.int32)
    src3 = jnp.concatenate([edge_index[0], pad]).reshape(16, _NCH, _EC)
    dst3 = jnp.concatenate([edge_index[1], pad]).reshape(16, _NCH, _EC)
    xp = jnp.pad(x, ((0, _NP - _N), (0, 0)))

    deg2 = _deg_kernel(dst3)
    zt0, dinvb = _pre(xp, Wc0, deg2)
    acc1 = _conv_kernel(zt0, src3, dst3)
    zt1, y1 = _mid(acc1, zt0, dinvb, bc0.reshape(1, -1), Wc1)
    acc2 = _conv_kernel(zt1, src3, dst3)
    zt2, _ = _mid(acc2, zt1, dinvb, bc1.reshape(1, -1), Wc2, res=y1)
    acc3 = _conv_kernel(zt2, src3, dst3)
    outp = _final(acc3, zt2, dinvb, bc2.reshape(1, -1), y1,
                  Wf1, bf1.reshape(1, -1), Wf2, bf2.reshape(1, -1),
                  jnp.pad(Wf3, ((0, 0), (0, 128 - 7))),
                  jnp.pad(bf3, (0, 128 - 7)).reshape(1, 128))
    return outp[:_N, :7]
